# Initial kernel scaffold; baseline (speedup 1.0000x reference)
import jax
import jax.numpy as jnp
from jax.experimental import pallas as pl


def kernel(x, edge_index, batch, W1, b1, W2, b2):
    raise NotImplementedError("write your pallas kernel here")


# speedup vs baseline: 10.3146x; 10.3146x over previous
import functools

import jax
import jax.numpy as jnp
from jax import lax
from jax.experimental import pallas as pl
from jax.experimental.pallas import tpu as pltpu
from jax.experimental.pallas import tpu_sc as plsc

NC = 2
NS = 16
NW = NC * NS
CHUNK = 128


def _make_deg_kernel(n_pad, e_pad):
    ept = e_pad // NW
    n_chunks = ept // CHUNK
    rows_pc = n_pad // NS
    mesh = plsc.VectorSubcoreMesh(core_axis_name="c", subcore_axis_name="s")

    @functools.partial(
        pl.kernel,
        mesh=mesh,
        out_type=jax.ShapeDtypeStruct((NC, n_pad, 128), jnp.float32),
        scratch_types=[
            pltpu.VMEM((CHUNK,), jnp.int32),
            pltpu.VMEM((CHUNK, 128), jnp.float32),
            pltpu.VMEM_SHARED((n_pad, 128), jnp.float32),
        ],
    )
    def deg_kernel(dst_hbm, ones_hbm, zeros_hbm, out_hbm, idx_v, ones_v, accum):
        cid = lax.axis_index("c")
        sid = lax.axis_index("s")
        wid = cid * NS + sid
        pltpu.sync_copy(ones_hbm, ones_v)
        pltpu.sync_copy(zeros_hbm, accum.at[pl.ds(sid * rows_pc, rows_pc)])
        plsc.subcore_barrier()

        @pl.loop(0, n_chunks)
        def _(i):
            off = wid * ept + i * CHUNK
            pltpu.sync_copy(dst_hbm.at[pl.ds(off, CHUNK)], idx_v)
            pltpu.sync_copy(ones_v, accum.at[idx_v], add=True)

        plsc.subcore_barrier()
        pltpu.sync_copy(
            accum.at[pl.ds(sid * rows_pc, rows_pc)],
            out_hbm.at[cid, pl.ds(sid * rows_pc, rows_pc)],
        )

    return deg_kernel


def _make_msg_kernel(n_pad, e_pad, d):
    ept = e_pad // NW
    n_chunks = ept // CHUNK
    rows_pc = n_pad // NS
    mesh = plsc.VectorSubcoreMesh(core_axis_name="c", subcore_axis_name="s")

    @functools.partial(
        pl.kernel,
        mesh=mesh,
        out_type=jax.ShapeDtypeStruct((NC, n_pad, d), jnp.float32),
        scratch_types=[
            pltpu.VMEM((CHUNK,), jnp.int32),
            pltpu.VMEM((CHUNK,), jnp.int32),
            pltpu.VMEM((CHUNK, d), jnp.float32),
            pltpu.VMEM_SHARED((n_pad, d), jnp.float32),
            pltpu.SemaphoreType.DMA,
        ],
    )
    def msg_kernel(y_hbm, src_hbm, dst_hbm, zeros_hbm, out_hbm,
                   src_v, dst_v, rows_v, accum, sem):
        cid = lax.axis_index("c")
        sid = lax.axis_index("s")
        wid = cid * NS + sid
        pltpu.sync_copy(zeros_hbm, accum.at[pl.ds(sid * rows_pc, rows_pc)])
        plsc.subcore_barrier()

        @pl.loop(0, n_chunks)
        def _(i):
            off = wid * ept + i * CHUNK
            pltpu.sync_copy(src_hbm.at[pl.ds(off, CHUNK)], src_v)
            pltpu.sync_copy(dst_hbm.at[pl.ds(off, CHUNK)], dst_v)
            pltpu.async_copy(y_hbm.at[src_v], rows_v, sem).wait()
            pltpu.sync_copy(rows_v, accum.at[dst_v], add=True)

        plsc.subcore_barrier()
        pltpu.sync_copy(
            accum.at[pl.ds(sid * rows_pc, rows_pc)],
            out_hbm.at[cid, pl.ds(sid * rows_pc, rows_pc)],
        )

    return msg_kernel


def _tc_xw(x_pad, w):
    def body(x_ref, w_ref, o_ref):
        o_ref[...] = jnp.dot(x_ref[...], w_ref[...],
                             preferred_element_type=jnp.float32)

    return pl.pallas_call(
        body,
        out_shape=jax.ShapeDtypeStruct((x_pad.shape[0], w.shape[1]), jnp.float32),
    )(x_pad, w)


def _tc_norm_scale(degp, xw):
    n_pad, d = xw.shape

    def body(degp_ref, xw_ref, dinv_ref, y_ref):
        deg = degp_ref[0, :, 0:1] + degp_ref[1, :, 0:1] + 1.0
        dinv = lax.rsqrt(deg)
        dinv_ref[...] = dinv
        y_ref[...] = xw_ref[...] * dinv

    return pl.pallas_call(
        body,
        out_shape=(
            jax.ShapeDtypeStruct((n_pad, 1), jnp.float32),
            jax.ShapeDtypeStruct((n_pad, d), jnp.float32),
        ),
    )(degp, xw)


def _tc_layer2(parts, y, dinv, b, w):
    n_pad = y.shape[0]
    d_out = w.shape[1]

    def body(p_ref, y_ref, dinv_ref, b_ref, w_ref, y2_ref):
        s = p_ref[0] + p_ref[1] + y_ref[...]
        h = jnp.maximum(s * dinv_ref[...] + b_ref[...], 0.0)
        y2_ref[...] = jnp.dot(h, w_ref[...],
                              preferred_element_type=jnp.float32) * dinv_ref[...]

    return pl.pallas_call(
        body,
        out_shape=jax.ShapeDtypeStruct((n_pad, d_out), jnp.float32),
    )(parts, y, dinv, b, w)


def _tc_finish(parts, y, dinv, b, batch2d, g):
    n_pad, d = y.shape

    d_out = b.shape[1]

    def body(p_ref, y_ref, dinv_ref, b_ref, batch_ref, o_ref):
        s = (p_ref[0] + p_ref[1] + y_ref[...])[:, :d_out]
        h = jnp.maximum(s * dinv_ref[...] + b_ref[...], 0.0)
        gids = lax.broadcasted_iota(jnp.int32, (g, n_pad), 0)
        mask = (gids == batch_ref[...]).astype(jnp.float32)
        o_ref[...] = jnp.dot(mask, h, preferred_element_type=jnp.float32)

    return pl.pallas_call(
        body,
        out_shape=jax.ShapeDtypeStruct((g, d_out), jnp.float32),
    )(parts, y, dinv, b, batch2d)


def kernel(x, edge_index, batch, W1, b1, W2, b2):
    n, d_in = x.shape
    e = edge_index.shape[1]
    d_hid = W1.shape[1]
    d_out = W2.shape[1]
    g = 16

    n_pad = ((n + 1 + 127) // 128) * 128
    e_pad = ((e + NW * CHUNK - 1) // (NW * CHUNK)) * (NW * CHUNK)
    rows_pc = n_pad // NS

    src = jnp.concatenate(
        [edge_index[0], jnp.zeros((e_pad - e,), dtype=jnp.int32)])
    dst = jnp.concatenate(
        [edge_index[1], jnp.full((e_pad - e,), n, dtype=jnp.int32)])
    x_pad = jnp.pad(x, ((0, n_pad - n), (0, 0)))
    batch2d = jnp.pad(batch, (0, n_pad - n),
                      constant_values=g).reshape(1, n_pad)
    ones128 = jnp.ones((CHUNK, 128), jnp.float32)
    zeros_m1 = jnp.zeros((rows_pc, d_hid), jnp.float32)
    w2_pad = jnp.pad(W2, ((0, 0), (0, d_hid - d_out)))

    degp = _make_deg_kernel(n_pad, e_pad)(dst, ones128, zeros_m1)
    xw1 = _tc_xw(x_pad, W1)

    dinv, y1 = _tc_norm_scale(degp, xw1)
    parts1 = _make_msg_kernel(n_pad, e_pad, d_hid)(y1, src, dst, zeros_m1)
    y2 = _tc_layer2(parts1, y1, dinv, b1.reshape(1, d_hid), w2_pad)
    parts2 = _make_msg_kernel(n_pad, e_pad, d_hid)(y2, src, dst, zeros_m1)
    return _tc_finish(parts2, y2, dinv, b2.reshape(1, d_out), batch2d, g)
